# baseline (speedup 1.0000x reference)
import jax
import jax.numpy as jnp
from jax import lax
from jax.experimental import pallas as pl
from jax.experimental.pallas import tpu as pltpu
from jax.experimental.pallas import tpu_sc as plsc

BATCH = 4096
HIST = 200
EMB_DIM = 64
NUM_Y = 16
NC = 2
NS = 16
NW = NC * NS
BAGS_PER_W = BATCH // NW
SPLIT = 104


def _sc_bag_sum(text_hbm, emb_hbm, out_hbm,
                idx_v, buf0, buf1, out_v, sem0, sem1, lin_sem):
    wid = lax.axis_index("s") * NC + lax.axis_index("c")
    base = wid * BAGS_PER_W

    cp = pltpu.make_async_copy(text_hbm.at[pl.ds(base, BAGS_PER_W)], idx_v,
                               lin_sem)
    cp.start()
    cp.wait()

    bufs = (buf0, buf1)
    sems = (sem0, sem1)

    def issue(bag, buf, sem):
        pltpu.make_async_copy(
            emb_hbm.at[idx_v.at[bag, pl.ds(0, SPLIT)]],
            buf.at[pl.ds(0, SPLIT)], sem).start()
        pltpu.make_async_copy(
            emb_hbm.at[idx_v.at[bag, pl.ds(SPLIT, HIST - SPLIT)]],
            buf.at[pl.ds(SPLIT, HIST - SPLIT)], sem).start()

    def wait(buf, sem):
        pltpu.make_async_copy(emb_hbm.at[pl.ds(0, HIST)], buf, sem).wait()

    def compute(bag, buf):
        def acc_body(r, acc):
            a0, a1, a2, a3 = acc
            return (a0 + buf[r, pl.ds(0, 16)],
                    a1 + buf[r, pl.ds(16, 16)],
                    a2 + buf[r, pl.ds(32, 16)],
                    a3 + buf[r, pl.ds(48, 16)])
        z = jnp.zeros((16,), jnp.float32)
        a0, a1, a2, a3 = lax.fori_loop(0, HIST, acc_body, (z, z, z, z))
        out_v[bag, pl.ds(0, 16)] = a0
        out_v[bag, pl.ds(16, 16)] = a1
        out_v[bag, pl.ds(32, 16)] = a2
        out_v[bag, pl.ds(48, 16)] = a3

    issue(0, bufs[0], sems[0])

    def outer(i, carry):
        g = 2 * i
        issue(g + 1, bufs[1], sems[1])
        wait(bufs[0], sems[0])
        compute(g, bufs[0])

        @pl.when(g + 2 < BAGS_PER_W)
        def _():
            issue(g + 2, bufs[0], sems[0])
        wait(bufs[1], sems[1])
        compute(g + 1, bufs[1])
        return carry

    lax.fori_loop(0, BAGS_PER_W // 2, outer, 0)

    cp = pltpu.make_async_copy(out_v, out_hbm.at[pl.ds(base, BAGS_PER_W)],
                               lin_sem)
    cp.start()
    cp.wait()


def _tc_linear(sums_ref, w_ref, b_ref, out_ref):
    out_ref[...] = (
        jnp.dot(sums_ref[...], w_ref[...], preferred_element_type=jnp.float32)
        + b_ref[...]
    )


@jax.jit
def _run(text, emb_weight, w_scaled, lin_b):
    mesh = plsc.VectorSubcoreMesh(core_axis_name="c", subcore_axis_name="s")
    bag_sums = pl.kernel(
        _sc_bag_sum,
        out_type=jax.ShapeDtypeStruct((BATCH, EMB_DIM), jnp.float32),
        mesh=mesh,
        scratch_types=[
            pltpu.VMEM((BAGS_PER_W, HIST), jnp.int32),
            pltpu.VMEM((HIST, EMB_DIM), jnp.float32),
            pltpu.VMEM((HIST, EMB_DIM), jnp.float32),
            pltpu.VMEM((BAGS_PER_W, EMB_DIM), jnp.float32),
            pltpu.SemaphoreType.DMA,
            pltpu.SemaphoreType.DMA,
            pltpu.SemaphoreType.DMA,
        ],
        compiler_params=pltpu.CompilerParams(use_tc_tiling_on_sc=False),
    )(text, emb_weight)
    return pl.pallas_call(
        _tc_linear,
        out_shape=jax.ShapeDtypeStruct((BATCH, NUM_Y), jnp.float32),
    )(bag_sums, w_scaled, lin_b.reshape(1, NUM_Y))


def kernel(text, emb_weight, lin_w, lin_b):
    w_scaled = lin_w.T.reshape(EMB_DIM, NUM_Y) * jnp.float32(1.0 / HIST)
    return _run(text.astype(jnp.int32), emb_weight, w_scaled, lin_b)
